# Initial kernel scaffold; baseline (speedup 1.0000x reference)
import jax
import jax.numpy as jnp
from jax.experimental import pallas as pl


def kernel(x_pfas_sites, x_gw_wells, x_sw_stations, edge_index_pfas_to_gw, edge_index_gw_to_pfas, edge_index_pfas_to_sw, edge_index_sw_to_pfas, Wl_pg, bl_pg, Wr_pg, Wl_gp, bl_gp, Wr_gp, Wl_ps, bl_ps, Wr_ps, Wl_sp, bl_sp, Wr_sp, W_gw, b_gw, W_sw, b_sw):
    raise NotImplementedError("write your pallas kernel here")


# speedup vs baseline: 2.6823x; 2.6823x over previous
import functools

import jax
import jax.numpy as jnp
from jax import lax
from jax.experimental import pallas as pl
from jax.experimental.pallas import tpu as pltpu
from jax.experimental.pallas import tpu_sc as plsc

N = 10000
D = 256
H = 128
E = 160000
NC = 2
NS = 16
CH = 128
EPT = 10112
E_PAD = NS * EPT
NCHUNK = EPT // CH
NACC = 10240
ZPT = NACC // NS
RPT = 640
RPT_LAST = N - (NS - 1) * RPT

_f32 = jnp.float32
_i32 = jnp.int32


def _sc_body(t_pf, t_gw, t_sw,
             src2_pg, src2_gp, src2_ps, src2_sp,
             dst_pg, dst_gp, dst_ps, dst_sp,
             zrow, zcnt, onesh,
             s_pg, s_gp, s_ps, s_sp,
             c_pg, c_gp, c_ps, c_sp,
             iss, isd, rows, onev, acc, cnt, gsem):
    c = lax.axis_index("c")
    s = lax.axis_index("s")
    pltpu.sync_copy(onesh, onev)

    rels = ((t_pf, src2_pg, dst_pg, s_pg, c_pg),
            (t_gw, src2_gp, dst_gp, s_gp, c_gp),
            (t_pf, src2_ps, dst_ps, s_ps, c_ps),
            (t_sw, src2_sp, dst_sp, s_sp, c_sp))

    for tab, src2, dsth, s_out, c_out in rels:
        pltpu.sync_copy(zrow, acc.at[pl.ds(s * ZPT, ZPT)])

        @pl.when(c == 0)
        def _zero_cnt():
            pltpu.sync_copy(zcnt.at[pl.ds(s * ZPT, ZPT)],
                            cnt.at[pl.ds(s * ZPT, ZPT)])

        plsc.subcore_barrier()

        def _step(j, carry):
            pltpu.sync_copy(src2.at[c, pl.ds(s * EPT + j * CH, CH)], iss)
            pltpu.sync_copy(dsth.at[pl.ds(s * EPT + j * CH, CH)], isd)
            pltpu.async_copy(tab.at[iss], rows, gsem).wait()
            pltpu.sync_copy(rows, acc.at[isd], add=True)

            @pl.when(c == 0)
            def _count():
                pltpu.sync_copy(onev, cnt.at[isd], add=True)

            return carry

        lax.fori_loop(0, NCHUNK, _step, 0)
        plsc.subcore_barrier()

        @pl.when(s < NS - 1)
        def _dump():
            pltpu.sync_copy(acc.at[pl.ds(s * RPT, RPT)],
                            s_out.at[pl.ds(c * N + s * RPT, RPT)])

        @pl.when(s == NS - 1)
        def _dump_last():
            pltpu.sync_copy(acc.at[pl.ds((NS - 1) * RPT, RPT_LAST)],
                            s_out.at[pl.ds(c * N + (NS - 1) * RPT, RPT_LAST)])

        @pl.when(c == 0)
        def _dump_cnt():
            pltpu.sync_copy(cnt.at[pl.ds(s * ZPT, ZPT)],
                            c_out.at[pl.ds(s * ZPT, ZPT)])


_sc_segsum = functools.partial(
    pl.kernel,
    out_type=[jax.ShapeDtypeStruct((2 * N, H), _f32) for _ in range(4)]
    + [jax.ShapeDtypeStruct((NACC,), _f32) for _ in range(4)],
    mesh=plsc.VectorSubcoreMesh(
        core_axis_name="c", subcore_axis_name="s", num_cores=NC, num_subcores=NS),
    scratch_types=[
        pltpu.VMEM((CH,), _i32),
        pltpu.VMEM((CH,), _i32),
        pltpu.VMEM((CH, H), _f32),
        pltpu.VMEM((CH,), _f32),
        pltpu.VMEM_SHARED((NACC, H), _f32),
        pltpu.VMEM_SHARED((NACC,), _f32),
        pltpu.SemaphoreType.DMA,
    ],
)(_sc_body)


def _tc_body(xp, xg, xs,
             sgp0, sgp1, ssp0, ssp1, spg0, spg1, sps0, sps1,
             cgp, csp, cpg, cps,
             wlgp_a, wlgp_b, wlsp_a, wlsp_b, wlpg_a, wlpg_b, wlps_a, wlps_b,
             wrgp, wrsp, wrpg, wrps,
             blgp, blsp, blpg, blps,
             wgw, bgw, wsw, bsw,
             pfas_o, gw_o, sw_o):
    def dot(a, b):
        return jnp.dot(a, b, preferred_element_type=_f32)

    r_gp = 1.0 / jnp.maximum(cgp[...], 1.0)
    r_sp = 1.0 / jnp.maximum(csp[...], 1.0)
    r_pg = 1.0 / jnp.maximum(cpg[...], 1.0)
    r_ps = 1.0 / jnp.maximum(cps[...], 1.0)

    o_pf = (dot(sgp0[...] * r_gp, wlgp_a[...]) + dot(sgp1[...] * r_gp, wlgp_b[...])
            + dot(ssp0[...] * r_sp, wlsp_a[...]) + dot(ssp1[...] * r_sp, wlsp_b[...])
            + dot(xp[...], wrgp[...] + wrsp[...])
            + blgp[...] + blsp[...])
    pfas_o[...] = jnp.maximum(o_pf, 0.0)

    o_gw = (dot(spg0[...] * r_pg, wlpg_a[...]) + dot(spg1[...] * r_pg, wlpg_b[...])
            + dot(xg[...], wrpg[...]) + blpg[...])
    gw_o[...] = dot(jnp.maximum(o_gw, 0.0), wgw[...]) + bgw[...]

    o_sw = (dot(sps0[...] * r_ps, wlps_a[...]) + dot(sps1[...] * r_ps, wlps_b[...])
            + dot(xs[...], wrps[...]) + blps[...])
    sw_o[...] = dot(jnp.maximum(o_sw, 0.0), wsw[...]) + bsw[...]


_R = 1000
_G = N // _R


def _tc_dense(xp, xg, xs, s_gp, s_sp, s_pg, s_ps, c_gp, c_sp, c_pg, c_ps,
              wlgp_a, wlgp_b, wlsp_a, wlsp_b, wlpg_a, wlpg_b, wlps_a, wlps_b,
              wrgp, wrsp, wrpg, wrps, blgp, blsp, blpg, blps,
              wgw, bgw, wsw, bsw):
    row = pl.BlockSpec((_R, D), lambda i: (i, 0))
    half_lo = pl.BlockSpec((_R, H), lambda i: (i, 0))
    half_hi = pl.BlockSpec((_R, H), lambda i: (i + N // _R, 0))
    cntb = pl.BlockSpec((_R, 1), lambda i: (i, 0))
    whalf = pl.BlockSpec((H, D), lambda i: (0, 0))
    wfull = pl.BlockSpec((D, D), lambda i: (0, 0))
    brow = pl.BlockSpec((1, D), lambda i: (0, 0))
    wcol = pl.BlockSpec((D, 1), lambda i: (0, 0))
    bone = pl.BlockSpec((1, 1), lambda i: (0, 0))

    return pl.pallas_call(
        _tc_body,
        grid=(_G,),
        in_specs=[row, row, row,
                  half_lo, half_hi, half_lo, half_hi,
                  half_lo, half_hi, half_lo, half_hi,
                  cntb, cntb, cntb, cntb,
                  whalf, whalf, whalf, whalf, whalf, whalf, whalf, whalf,
                  wfull, wfull, wfull, wfull,
                  brow, brow, brow, brow,
                  wcol, bone, wcol, bone],
        out_specs=[pl.BlockSpec((_R, D), lambda i: (i, 0)),
                   pl.BlockSpec((_R, 1), lambda i: (i, 0)),
                   pl.BlockSpec((_R, 1), lambda i: (i, 0))],
        out_shape=[jax.ShapeDtypeStruct((N, D), _f32),
                   jax.ShapeDtypeStruct((N, 1), _f32),
                   jax.ShapeDtypeStruct((N, 1), _f32)],
    )(xp, xg, xs,
      s_gp, s_gp, s_sp, s_sp, s_pg, s_pg, s_ps, s_ps,
      c_gp, c_sp, c_pg, c_ps,
      wlgp_a, wlgp_b, wlsp_a, wlsp_b, wlpg_a, wlpg_b, wlps_a, wlps_b,
      wrgp, wrsp, wrpg, wrps, blgp, blsp, blpg, blps,
      wgw, bgw, wsw, bsw)


def _halves(x):
    return jnp.concatenate([x[:, :H], x[:, H:]], axis=0)


def _prep_edges(ei):
    pad = E_PAD - E
    src = jnp.concatenate([ei[0], jnp.zeros((pad,), _i32)])
    dst = jnp.concatenate([ei[1], jnp.full((pad,), N, _i32)])
    src2 = jnp.stack([src, src + N])
    return src2, dst


def kernel(x_pfas_sites, x_gw_wells, x_sw_stations,
           edge_index_pfas_to_gw, edge_index_gw_to_pfas,
           edge_index_pfas_to_sw, edge_index_sw_to_pfas,
           Wl_pg, bl_pg, Wr_pg,
           Wl_gp, bl_gp, Wr_gp,
           Wl_ps, bl_ps, Wr_ps,
           Wl_sp, bl_sp, Wr_sp,
           W_gw, b_gw, W_sw, b_sw):
    t_pf = _halves(x_pfas_sites)
    t_gw = _halves(x_gw_wells)
    t_sw = _halves(x_sw_stations)
    src2_pg, dst_pg = _prep_edges(edge_index_pfas_to_gw)
    src2_gp, dst_gp = _prep_edges(edge_index_gw_to_pfas)
    src2_ps, dst_ps = _prep_edges(edge_index_pfas_to_sw)
    src2_sp, dst_sp = _prep_edges(edge_index_sw_to_pfas)

    zrow = jnp.zeros((ZPT, H), _f32)
    zcnt = jnp.zeros((NACC,), _f32)
    onesh = jnp.ones((CH,), _f32)

    (s_pg, s_gp, s_ps, s_sp, c_pg, c_gp, c_ps, c_sp) = _sc_segsum(
        t_pf, t_gw, t_sw,
        src2_pg, src2_gp, src2_ps, src2_sp,
        dst_pg, dst_gp, dst_ps, dst_sp,
        zrow, zcnt, onesh)

    c_gp, c_sp, c_pg, c_ps = (c[:N].reshape(N, 1)
                              for c in (c_gp, c_sp, c_pg, c_ps))
    pfas, gw, sw = _tc_dense(
        x_pfas_sites, x_gw_wells, x_sw_stations,
        s_gp, s_sp, s_pg, s_ps, c_gp, c_sp, c_pg, c_ps,
        Wl_gp[:H], Wl_gp[H:], Wl_sp[:H], Wl_sp[H:],
        Wl_pg[:H], Wl_pg[H:], Wl_ps[:H], Wl_ps[H:],
        Wr_gp, Wr_sp, Wr_pg, Wr_ps,
        bl_gp.reshape(1, D), bl_sp.reshape(1, D),
        bl_pg.reshape(1, D), bl_ps.reshape(1, D),
        W_gw, b_gw.reshape(1, 1), W_sw, b_sw.reshape(1, 1))
    return (pfas, gw, sw)


# speedup vs baseline: 3.8363x; 1.4302x over previous
import functools

import jax
import jax.numpy as jnp
from jax import lax
from jax.experimental import pallas as pl
from jax.experimental.pallas import tpu as pltpu
from jax.experimental.pallas import tpu_sc as plsc

N = 10000
D = 256
H = 128
E = 160000
NC = 2
NS = 16
CH = 128
EPT = 10112
E_PAD = NS * EPT
NCHUNK = EPT // CH
NACC = 10240
ZPT = NACC // NS
RPT = 640
RPT_LAST = N - (NS - 1) * RPT

_f32 = jnp.float32
_i32 = jnp.int32


NP = (NCHUNK - 1) // 2


def _sc_body(t_pf, t_gw, t_sw,
             src2_pg, src2_gp, src2_ps, src2_sp,
             dst_pg, dst_gp, dst_ps, dst_sp,
             zrow, zcnt, onesh,
             s_pg, s_gp, s_ps, s_sp,
             c_pg, c_gp, c_ps, c_sp,
             isrc, isdA, isdB, rowsA, rowsB, onev, acc, cnt,
             gsemA, gsemB, ssemA, ssemB, csemA, csemB):
    c = lax.axis_index("c")
    s = lax.axis_index("s")
    pltpu.sync_copy(onesh, onev)

    rels = ((t_pf, src2_pg, dst_pg, s_pg, c_pg, 0),
            (t_gw, src2_gp, dst_gp, s_gp, c_gp, 0),
            (t_pf, src2_ps, dst_ps, s_ps, c_ps, 1),
            (t_sw, src2_sp, dst_sp, s_sp, c_sp, 1))

    for tab, src2, dsth, s_out, c_out, cc in rels:
        pltpu.sync_copy(zrow, acc.at[pl.ds(s * ZPT, ZPT)])

        @pl.when(c == cc)
        def _zero_cnt():
            pltpu.sync_copy(zcnt.at[pl.ds(s * ZPT, ZPT)],
                            cnt.at[pl.ds(s * ZPT, ZPT)])

        pltpu.sync_copy(src2.at[c, pl.ds(s * EPT, EPT)], isrc)
        plsc.subcore_barrier()

        def _chunk(j, isd, rows, gsem, ssem, csem, drain):
            if drain is None:
                pltpu.make_async_copy(rows, acc.at[isd], ssem).wait()
            else:
                @pl.when(drain)
                def _():
                    pltpu.make_async_copy(rows, acc.at[isd], ssem).wait()

            cdrain = (c == cc) if drain is None else jnp.logical_and(drain, c == cc)

            @pl.when(cdrain)
            def _():
                pltpu.make_async_copy(onev, cnt.at[isd], csem).wait()

            gd = pltpu.async_copy(tab.at[isrc.at[pl.ds(j * CH, CH)]], rows, gsem)
            pltpu.sync_copy(dsth.at[pl.ds(s * EPT + j * CH, CH)], isd)
            gd.wait()
            pltpu.async_copy(rows, acc.at[isd], ssem, add=True)

            @pl.when(c == cc)
            def _count():
                pltpu.async_copy(onev, cnt.at[isd], csem, add=True)

        def _pair(i, carry):
            _chunk(2 * i, isdA, rowsA, gsemA, ssemA, csemA, i > 0)
            _chunk(2 * i + 1, isdB, rowsB, gsemB, ssemB, csemB, i > 0)
            return carry

        lax.fori_loop(0, NP, _pair, 0)
        _chunk(NCHUNK - 1, isdA, rowsA, gsemA, ssemA, csemA, None)

        pltpu.make_async_copy(rowsA, acc.at[isdA], ssemA).wait()
        pltpu.make_async_copy(rowsB, acc.at[isdB], ssemB).wait()

        @pl.when(c == cc)
        def _final_cnt():
            pltpu.make_async_copy(onev, cnt.at[isdA], csemA).wait()
            pltpu.make_async_copy(onev, cnt.at[isdB], csemB).wait()

        plsc.subcore_barrier()

        @pl.when(s < NS - 1)
        def _dump():
            pltpu.sync_copy(acc.at[pl.ds(s * RPT, RPT)],
                            s_out.at[pl.ds(c * N + s * RPT, RPT)])

        @pl.when(s == NS - 1)
        def _dump_last():
            pltpu.sync_copy(acc.at[pl.ds((NS - 1) * RPT, RPT_LAST)],
                            s_out.at[pl.ds(c * N + (NS - 1) * RPT, RPT_LAST)])

        @pl.when(c == cc)
        def _dump_cnt():
            pltpu.sync_copy(cnt.at[pl.ds(s * ZPT, ZPT)],
                            c_out.at[pl.ds(s * ZPT, ZPT)])


_sc_segsum = functools.partial(
    pl.kernel,
    out_type=[jax.ShapeDtypeStruct((2 * N, H), _f32) for _ in range(4)]
    + [jax.ShapeDtypeStruct((NACC,), _f32) for _ in range(4)],
    mesh=plsc.VectorSubcoreMesh(
        core_axis_name="c", subcore_axis_name="s", num_cores=NC, num_subcores=NS),
    scratch_types=[
        pltpu.VMEM((EPT,), _i32),
        pltpu.VMEM((CH,), _i32),
        pltpu.VMEM((CH,), _i32),
        pltpu.VMEM((CH, H), _f32),
        pltpu.VMEM((CH, H), _f32),
        pltpu.VMEM((CH,), _f32),
        pltpu.VMEM_SHARED((NACC, H), _f32),
        pltpu.VMEM_SHARED((NACC,), _f32),
        pltpu.SemaphoreType.DMA,
        pltpu.SemaphoreType.DMA,
        pltpu.SemaphoreType.DMA,
        pltpu.SemaphoreType.DMA,
        pltpu.SemaphoreType.DMA,
        pltpu.SemaphoreType.DMA,
    ],
)(_sc_body)


def _tc_body(xp, xg, xs,
             sgp0, sgp1, ssp0, ssp1, spg0, spg1, sps0, sps1,
             cgp, csp, cpg, cps,
             wlgp_a, wlgp_b, wlsp_a, wlsp_b, wlpg_a, wlpg_b, wlps_a, wlps_b,
             wrgp, wrsp, wrpg, wrps,
             blgp, blsp, blpg, blps,
             wgw, bgw, wsw, bsw,
             pfas_o, gw_o, sw_o):
    def dot(a, b):
        return jnp.dot(a, b, preferred_element_type=_f32)

    r_gp = 1.0 / jnp.maximum(cgp[...], 1.0)
    r_sp = 1.0 / jnp.maximum(csp[...], 1.0)
    r_pg = 1.0 / jnp.maximum(cpg[...], 1.0)
    r_ps = 1.0 / jnp.maximum(cps[...], 1.0)

    o_pf = (dot(sgp0[...] * r_gp, wlgp_a[...]) + dot(sgp1[...] * r_gp, wlgp_b[...])
            + dot(ssp0[...] * r_sp, wlsp_a[...]) + dot(ssp1[...] * r_sp, wlsp_b[...])
            + dot(xp[...], wrgp[...] + wrsp[...])
            + blgp[...] + blsp[...])
    pfas_o[...] = jnp.maximum(o_pf, 0.0)

    o_gw = (dot(spg0[...] * r_pg, wlpg_a[...]) + dot(spg1[...] * r_pg, wlpg_b[...])
            + dot(xg[...], wrpg[...]) + blpg[...])
    gw_o[...] = dot(jnp.maximum(o_gw, 0.0), wgw[...]) + bgw[...]

    o_sw = (dot(sps0[...] * r_ps, wlps_a[...]) + dot(sps1[...] * r_ps, wlps_b[...])
            + dot(xs[...], wrps[...]) + blps[...])
    sw_o[...] = dot(jnp.maximum(o_sw, 0.0), wsw[...]) + bsw[...]


_R = 1000
_G = N // _R


def _tc_dense(xp, xg, xs, s_gp, s_sp, s_pg, s_ps, c_gp, c_sp, c_pg, c_ps,
              wlgp_a, wlgp_b, wlsp_a, wlsp_b, wlpg_a, wlpg_b, wlps_a, wlps_b,
              wrgp, wrsp, wrpg, wrps, blgp, blsp, blpg, blps,
              wgw, bgw, wsw, bsw):
    row = pl.BlockSpec((_R, D), lambda i: (i, 0))
    half_lo = pl.BlockSpec((_R, H), lambda i: (i, 0))
    half_hi = pl.BlockSpec((_R, H), lambda i: (i + N // _R, 0))
    cntb = pl.BlockSpec((_R, 1), lambda i: (i, 0))
    whalf = pl.BlockSpec((H, D), lambda i: (0, 0))
    wfull = pl.BlockSpec((D, D), lambda i: (0, 0))
    brow = pl.BlockSpec((1, D), lambda i: (0, 0))
    wcol = pl.BlockSpec((D, 1), lambda i: (0, 0))
    bone = pl.BlockSpec((1, 1), lambda i: (0, 0))

    return pl.pallas_call(
        _tc_body,
        grid=(_G,),
        in_specs=[row, row, row,
                  half_lo, half_hi, half_lo, half_hi,
                  half_lo, half_hi, half_lo, half_hi,
                  cntb, cntb, cntb, cntb,
                  whalf, whalf, whalf, whalf, whalf, whalf, whalf, whalf,
                  wfull, wfull, wfull, wfull,
                  brow, brow, brow, brow,
                  wcol, bone, wcol, bone],
        out_specs=[pl.BlockSpec((_R, D), lambda i: (i, 0)),
                   pl.BlockSpec((_R, 1), lambda i: (i, 0)),
                   pl.BlockSpec((_R, 1), lambda i: (i, 0))],
        out_shape=[jax.ShapeDtypeStruct((N, D), _f32),
                   jax.ShapeDtypeStruct((N, 1), _f32),
                   jax.ShapeDtypeStruct((N, 1), _f32)],
    )(xp, xg, xs,
      s_gp, s_gp, s_sp, s_sp, s_pg, s_pg, s_ps, s_ps,
      c_gp, c_sp, c_pg, c_ps,
      wlgp_a, wlgp_b, wlsp_a, wlsp_b, wlpg_a, wlpg_b, wlps_a, wlps_b,
      wrgp, wrsp, wrpg, wrps, blgp, blsp, blpg, blps,
      wgw, bgw, wsw, bsw)


def _halves(x):
    return jnp.concatenate([x[:, :H], x[:, H:]], axis=0)


def _prep_edges(ei):
    pad = E_PAD - E
    src = jnp.concatenate([ei[0], jnp.zeros((pad,), _i32)])
    dst = jnp.concatenate([ei[1], jnp.full((pad,), N, _i32)])
    src2 = jnp.stack([src, src + N])
    return src2, dst


def kernel(x_pfas_sites, x_gw_wells, x_sw_stations,
           edge_index_pfas_to_gw, edge_index_gw_to_pfas,
           edge_index_pfas_to_sw, edge_index_sw_to_pfas,
           Wl_pg, bl_pg, Wr_pg,
           Wl_gp, bl_gp, Wr_gp,
           Wl_ps, bl_ps, Wr_ps,
           Wl_sp, bl_sp, Wr_sp,
           W_gw, b_gw, W_sw, b_sw):
    t_pf = _halves(x_pfas_sites)
    t_gw = _halves(x_gw_wells)
    t_sw = _halves(x_sw_stations)
    src2_pg, dst_pg = _prep_edges(edge_index_pfas_to_gw)
    src2_gp, dst_gp = _prep_edges(edge_index_gw_to_pfas)
    src2_ps, dst_ps = _prep_edges(edge_index_pfas_to_sw)
    src2_sp, dst_sp = _prep_edges(edge_index_sw_to_pfas)

    zrow = jnp.zeros((ZPT, H), _f32)
    zcnt = jnp.zeros((NACC,), _f32)
    onesh = jnp.ones((CH,), _f32)

    (s_pg, s_gp, s_ps, s_sp, c_pg, c_gp, c_ps, c_sp) = _sc_segsum(
        t_pf, t_gw, t_sw,
        src2_pg, src2_gp, src2_ps, src2_sp,
        dst_pg, dst_gp, dst_ps, dst_sp,
        zrow, zcnt, onesh)

    c_gp, c_sp, c_pg, c_ps = (c[:N].reshape(N, 1)
                              for c in (c_gp, c_sp, c_pg, c_ps))
    pfas, gw, sw = _tc_dense(
        x_pfas_sites, x_gw_wells, x_sw_stations,
        s_gp, s_sp, s_pg, s_ps, c_gp, c_sp, c_pg, c_ps,
        Wl_gp[:H], Wl_gp[H:], Wl_sp[:H], Wl_sp[H:],
        Wl_pg[:H], Wl_pg[H:], Wl_ps[:H], Wl_ps[H:],
        Wr_gp, Wr_sp, Wr_pg, Wr_ps,
        bl_gp.reshape(1, D), bl_sp.reshape(1, D),
        bl_pg.reshape(1, D), bl_ps.reshape(1, D),
        W_gw, b_gw.reshape(1, 1), W_sw, b_sw.reshape(1, 1))
    return (pfas, gw, sw)


# speedup vs baseline: 4.9382x; 1.2872x over previous
import functools

import jax
import jax.numpy as jnp
from jax import lax
from jax.experimental import pallas as pl
from jax.experimental.pallas import tpu as pltpu
from jax.experimental.pallas import tpu_sc as plsc

N = 10000
D = 256
H = 128
E = 160000
NC = 2
NS = 16
CH = 96
EPT = 10080
E_PAD = NS * EPT
NCHUNK = EPT // CH
NACC = 10240
ZPT = NACC // NS
RPT = 640
RPT_LAST = N - (NS - 1) * RPT

_f32 = jnp.float32
_i32 = jnp.int32


NT = NCHUNK // 3


def _sc_body(t_pf, t_gw, t_sw,
             src2_pg, src2_gp, src2_ps, src2_sp,
             dst_pg, dst_gp, dst_ps, dst_sp,
             zrow, zcnt, onesh,
             s_pg, s_gp, s_ps, s_sp,
             c_pg, c_gp, c_ps, c_sp,
             isrc, isd0, isd1, isd2, rows0, rows1, rows2, onev, acc, cnt,
             gsem0, gsem1, gsem2, ssem0, ssem1, ssem2,
             csem0, csem1, csem2, isem0, isem1, isem2):
    c = lax.axis_index("c")
    s = lax.axis_index("s")
    pltpu.sync_copy(onesh, onev)

    isd = (isd0, isd1, isd2)
    rows = (rows0, rows1, rows2)
    gsem = (gsem0, gsem1, gsem2)
    ssem = (ssem0, ssem1, ssem2)
    csem = (csem0, csem1, csem2)
    isem = (isem0, isem1, isem2)

    rels = ((t_pf, src2_pg, dst_pg, s_pg, c_pg, 0),
            (t_gw, src2_gp, dst_gp, s_gp, c_gp, 0),
            (t_pf, src2_ps, dst_ps, s_ps, c_ps, 1),
            (t_sw, src2_sp, dst_sp, s_sp, c_sp, 1))

    for tab, src2, dsth, s_out, c_out, cc in rels:
        pltpu.sync_copy(src2.at[pl.ds(c * E_PAD + s * EPT, EPT)], isrc)

        def _issue(j, b):
            pltpu.async_copy(tab.at[isrc.at[pl.ds(j * CH, CH)]], rows[b], gsem[b])
            pltpu.async_copy(dsth.at[pl.ds(s * EPT + j * CH, CH)], isd[b], isem[b])

        def _drain(b, guard):
            def _go():
                pltpu.make_async_copy(rows[b], acc.at[isd[b]], ssem[b]).wait()

            def _go_cnt():
                pltpu.make_async_copy(onev, cnt.at[isd[b]], csem[b]).wait()

            if guard is None:
                _go()
                pl.when(c == cc)(_go_cnt)
            else:
                pl.when(guard)(_go)
                pl.when(jnp.logical_and(guard, c == cc))(_go_cnt)

        def _finish(j, b):
            pltpu.make_async_copy(dsth.at[pl.ds(s * EPT + j * CH, CH)],
                                  isd[b], isem[b]).wait()
            pltpu.make_async_copy(tab.at[isrc.at[pl.ds(j * CH, CH)]],
                                  rows[b], gsem[b]).wait()
            pltpu.async_copy(rows[b], acc.at[isd[b]], ssem[b], add=True)

            @pl.when(c == cc)
            def _count():
                pltpu.async_copy(onev, cnt.at[isd[b]], csem[b], add=True)

        _issue(0, 0)

        pltpu.sync_copy(zrow, acc.at[pl.ds(s * ZPT, ZPT)])

        @pl.when(c == cc)
        def _zero_cnt():
            pltpu.sync_copy(zcnt.at[pl.ds(s * ZPT, ZPT)],
                            cnt.at[pl.ds(s * ZPT, ZPT)])

        plsc.subcore_barrier()

        def _triple(i, carry):
            j = 3 * i
            _drain(1, i > 0)
            _issue(j + 1, 1)
            _finish(j, 0)
            _drain(2, i > 0)
            _issue(j + 2, 2)
            _finish(j + 1, 1)
            _drain(0, None)

            @pl.when(j + 3 < NCHUNK)
            def _next():
                _issue(j + 3, 0)

            _finish(j + 2, 2)
            return carry

        lax.fori_loop(0, NT, _triple, 0)

        _drain(1, None)
        _drain(2, None)

        plsc.subcore_barrier()

        @pl.when(s < NS - 1)
        def _dump():
            pltpu.sync_copy(acc.at[pl.ds(s * RPT, RPT)],
                            s_out.at[pl.ds(c * N + s * RPT, RPT)])

        @pl.when(s == NS - 1)
        def _dump_last():
            pltpu.sync_copy(acc.at[pl.ds((NS - 1) * RPT, RPT_LAST)],
                            s_out.at[pl.ds(c * N + (NS - 1) * RPT, RPT_LAST)])

        @pl.when(c == cc)
        def _dump_cnt():
            pltpu.sync_copy(cnt.at[pl.ds(s * ZPT, ZPT)],
                            c_out.at[pl.ds(s * ZPT, ZPT)])


_sc_segsum = functools.partial(
    pl.kernel,
    out_type=[jax.ShapeDtypeStruct((2 * N, H), _f32) for _ in range(4)]
    + [jax.ShapeDtypeStruct((NACC,), _f32) for _ in range(4)],
    mesh=plsc.VectorSubcoreMesh(
        core_axis_name="c", subcore_axis_name="s", num_cores=NC, num_subcores=NS),
    scratch_types=[
        pltpu.VMEM((EPT,), _i32),
        pltpu.VMEM((CH,), _i32),
        pltpu.VMEM((CH,), _i32),
        pltpu.VMEM((CH,), _i32),
        pltpu.VMEM((CH, H), _f32),
        pltpu.VMEM((CH, H), _f32),
        pltpu.VMEM((CH, H), _f32),
        pltpu.VMEM((CH,), _f32),
        pltpu.VMEM_SHARED((NACC, H), _f32),
        pltpu.VMEM_SHARED((NACC,), _f32),
    ] + [pltpu.SemaphoreType.DMA] * 12,
)(_sc_body)


def _tc_body(xp, xg, xs,
             sgp0, sgp1, ssp0, ssp1, spg0, spg1, sps0, sps1,
             cgp, csp, cpg, cps,
             wlgp_a, wlgp_b, wlsp_a, wlsp_b, wlpg_a, wlpg_b, wlps_a, wlps_b,
             wrgp, wrsp, wrpg, wrps,
             blgp, blsp, blpg, blps,
             wgw, bgw, wsw, bsw,
             pfas_o, gw_o, sw_o):
    def dot(a, b):
        return jnp.dot(a, b, preferred_element_type=_f32)

    r_gp = 1.0 / jnp.maximum(cgp[...], 1.0)
    r_sp = 1.0 / jnp.maximum(csp[...], 1.0)
    r_pg = 1.0 / jnp.maximum(cpg[...], 1.0)
    r_ps = 1.0 / jnp.maximum(cps[...], 1.0)

    o_pf = (dot(sgp0[...] * r_gp, wlgp_a[...]) + dot(sgp1[...] * r_gp, wlgp_b[...])
            + dot(ssp0[...] * r_sp, wlsp_a[...]) + dot(ssp1[...] * r_sp, wlsp_b[...])
            + dot(xp[...], wrgp[...] + wrsp[...])
            + blgp[...] + blsp[...])
    pfas_o[...] = jnp.maximum(o_pf, 0.0)

    o_gw = (dot(spg0[...] * r_pg, wlpg_a[...]) + dot(spg1[...] * r_pg, wlpg_b[...])
            + dot(xg[...], wrpg[...]) + blpg[...])
    gw_o[...] = dot(jnp.maximum(o_gw, 0.0), wgw[...]) + bgw[...]

    o_sw = (dot(sps0[...] * r_ps, wlps_a[...]) + dot(sps1[...] * r_ps, wlps_b[...])
            + dot(xs[...], wrps[...]) + blps[...])
    sw_o[...] = dot(jnp.maximum(o_sw, 0.0), wsw[...]) + bsw[...]


_R = 1000
_G = N // _R


def _tc_dense(xp, xg, xs, s_gp, s_sp, s_pg, s_ps, c_gp, c_sp, c_pg, c_ps,
              wlgp_a, wlgp_b, wlsp_a, wlsp_b, wlpg_a, wlpg_b, wlps_a, wlps_b,
              wrgp, wrsp, wrpg, wrps, blgp, blsp, blpg, blps,
              wgw, bgw, wsw, bsw):
    row = pl.BlockSpec((_R, D), lambda i: (i, 0))
    half_lo = pl.BlockSpec((_R, H), lambda i: (i, 0))
    half_hi = pl.BlockSpec((_R, H), lambda i: (i + N // _R, 0))
    cntb = pl.BlockSpec((_R, 1), lambda i: (i, 0))
    whalf = pl.BlockSpec((H, D), lambda i: (0, 0))
    wfull = pl.BlockSpec((D, D), lambda i: (0, 0))
    brow = pl.BlockSpec((1, D), lambda i: (0, 0))
    wcol = pl.BlockSpec((D, 1), lambda i: (0, 0))
    bone = pl.BlockSpec((1, 1), lambda i: (0, 0))

    return pl.pallas_call(
        _tc_body,
        grid=(_G,),
        in_specs=[row, row, row,
                  half_lo, half_hi, half_lo, half_hi,
                  half_lo, half_hi, half_lo, half_hi,
                  cntb, cntb, cntb, cntb,
                  whalf, whalf, whalf, whalf, whalf, whalf, whalf, whalf,
                  wfull, wfull, wfull, wfull,
                  brow, brow, brow, brow,
                  wcol, bone, wcol, bone],
        out_specs=[pl.BlockSpec((_R, D), lambda i: (i, 0)),
                   pl.BlockSpec((_R, 1), lambda i: (i, 0)),
                   pl.BlockSpec((_R, 1), lambda i: (i, 0))],
        out_shape=[jax.ShapeDtypeStruct((N, D), _f32),
                   jax.ShapeDtypeStruct((N, 1), _f32),
                   jax.ShapeDtypeStruct((N, 1), _f32)],
    )(xp, xg, xs,
      s_gp, s_gp, s_sp, s_sp, s_pg, s_pg, s_ps, s_ps,
      c_gp, c_sp, c_pg, c_ps,
      wlgp_a, wlgp_b, wlsp_a, wlsp_b, wlpg_a, wlpg_b, wlps_a, wlps_b,
      wrgp, wrsp, wrpg, wrps, blgp, blsp, blpg, blps,
      wgw, bgw, wsw, bsw)


def _halves(x):
    return jnp.concatenate([x[:, :H], x[:, H:]], axis=0)


def _prep_edges(ei):
    pad = E_PAD - E
    src = jnp.concatenate([ei[0], jnp.zeros((pad,), _i32)])
    dst = jnp.concatenate([ei[1], jnp.full((pad,), N, _i32)])
    src2 = jnp.concatenate([src, src + N])
    return src2, dst


def kernel(x_pfas_sites, x_gw_wells, x_sw_stations,
           edge_index_pfas_to_gw, edge_index_gw_to_pfas,
           edge_index_pfas_to_sw, edge_index_sw_to_pfas,
           Wl_pg, bl_pg, Wr_pg,
           Wl_gp, bl_gp, Wr_gp,
           Wl_ps, bl_ps, Wr_ps,
           Wl_sp, bl_sp, Wr_sp,
           W_gw, b_gw, W_sw, b_sw):
    t_pf = _halves(x_pfas_sites)
    t_gw = _halves(x_gw_wells)
    t_sw = _halves(x_sw_stations)
    src2_pg, dst_pg = _prep_edges(edge_index_pfas_to_gw)
    src2_gp, dst_gp = _prep_edges(edge_index_gw_to_pfas)
    src2_ps, dst_ps = _prep_edges(edge_index_pfas_to_sw)
    src2_sp, dst_sp = _prep_edges(edge_index_sw_to_pfas)

    zrow = jnp.zeros((ZPT, H), _f32)
    zcnt = jnp.zeros((NACC,), _f32)
    onesh = jnp.ones((CH,), _f32)

    (s_pg, s_gp, s_ps, s_sp, c_pg, c_gp, c_ps, c_sp) = _sc_segsum(
        t_pf, t_gw, t_sw,
        src2_pg, src2_gp, src2_ps, src2_sp,
        dst_pg, dst_gp, dst_ps, dst_sp,
        zrow, zcnt, onesh)

    c_gp, c_sp, c_pg, c_ps = (c[:N].reshape(N, 1)
                              for c in (c_gp, c_sp, c_pg, c_ps))
    pfas, gw, sw = _tc_dense(
        x_pfas_sites, x_gw_wells, x_sw_stations,
        s_gp, s_sp, s_pg, s_ps, c_gp, c_sp, c_pg, c_ps,
        Wl_gp[:H], Wl_gp[H:], Wl_sp[:H], Wl_sp[H:],
        Wl_pg[:H], Wl_pg[H:], Wl_ps[:H], Wl_ps[H:],
        Wr_gp, Wr_sp, Wr_pg, Wr_ps,
        bl_gp.reshape(1, D), bl_sp.reshape(1, D),
        bl_pg.reshape(1, D), bl_ps.reshape(1, D),
        W_gw, b_gw.reshape(1, 1), W_sw, b_sw.reshape(1, 1))
    return (pfas, gw, sw)


# speedup vs baseline: 5.1632x; 1.0456x over previous
import functools

import jax
import jax.numpy as jnp
from jax import lax
from jax.experimental import pallas as pl
from jax.experimental.pallas import tpu as pltpu
from jax.experimental.pallas import tpu_sc as plsc

N = 10000
D = 256
H = 128
E = 160000
NC = 2
NS = 16
CH = 96
EPT = 10080
E_PAD = NS * EPT
NCHUNK = EPT // CH
NACC = 10240
ZPT = NACC // NS
RPT = 640
RPT_LAST = N - (NS - 1) * RPT

_f32 = jnp.float32
_i32 = jnp.int32


NT = NCHUNK // 3


def _sc_body(t_pf, t_gw, t_sw,
             src2_pg, src2_gp, src2_ps, src2_sp,
             dst_pg, dst_gp, dst_ps, dst_sp,
             zrow, zcnt, onesh,
             s_pg, s_gp, s_ps, s_sp,
             c_pg, c_gp, c_ps, c_sp,
             isrc, isd0, isd1, isd2, rows0, rows1, rows2, onev, acc, cnt,
             gsem0, gsem1, gsem2, ssem0, ssem1, ssem2,
             csem0, csem1, csem2, isem0, isem1, isem2):
    c = lax.axis_index("c")
    s = lax.axis_index("s")
    pltpu.sync_copy(onesh, onev)

    isd = (isd0, isd1, isd2)
    rows = (rows0, rows1, rows2)
    gsem = (gsem0, gsem1, gsem2)
    ssem = (ssem0, ssem1, ssem2)
    csem = (csem0, csem1, csem2)
    isem = (isem0, isem1, isem2)

    rels = ((t_pf, src2_pg, dst_pg, s_pg, c_pg, 0),
            (t_gw, src2_gp, dst_gp, s_gp, c_gp, 0),
            (t_pf, src2_ps, dst_ps, s_ps, c_ps, 1),
            (t_sw, src2_sp, dst_sp, s_sp, c_sp, 1))

    for tab, src2, dsth, s_out, c_out, cc in rels:
        pltpu.sync_copy(src2.at[pl.ds(c * E_PAD + s * EPT, EPT)], isrc)

        def _issue(j, b):
            pltpu.async_copy(tab.at[isrc.at[pl.ds(j * CH, CH)]], rows[b], gsem[b])
            pltpu.async_copy(dsth.at[pl.ds(s * EPT + j * CH, CH)], isd[b], isem[b])

        def _drain(b, guard):
            def _go():
                pltpu.make_async_copy(rows[b], acc.at[isd[b]], ssem[b]).wait()

            def _go_cnt():
                pltpu.make_async_copy(onev, cnt.at[isd[b]], csem[b]).wait()

            if guard is None:
                _go()
                pl.when(c == cc)(_go_cnt)
            else:
                pl.when(guard)(_go)
                pl.when(jnp.logical_and(guard, c == cc))(_go_cnt)

        def _finish(j, b):
            pltpu.make_async_copy(dsth.at[pl.ds(s * EPT + j * CH, CH)],
                                  isd[b], isem[b]).wait()
            pltpu.make_async_copy(tab.at[isrc.at[pl.ds(j * CH, CH)]],
                                  rows[b], gsem[b]).wait()
            pltpu.async_copy(rows[b], acc.at[isd[b]], ssem[b], add=True)

            @pl.when(c == cc)
            def _count():
                pltpu.async_copy(onev, cnt.at[isd[b]], csem[b], add=True)

        _issue(0, 0)

        pltpu.sync_copy(zrow, acc.at[pl.ds(s * ZPT, ZPT)])

        @pl.when(c == cc)
        def _zero_cnt():
            pltpu.sync_copy(zcnt.at[pl.ds(s * ZPT, ZPT)],
                            cnt.at[pl.ds(s * ZPT, ZPT)])

        plsc.subcore_barrier()

        def _triple(i, carry):
            j = 3 * i
            _drain(1, i > 0)
            _issue(j + 1, 1)
            _finish(j, 0)
            _drain(2, i > 0)
            _issue(j + 2, 2)
            _finish(j + 1, 1)
            _drain(0, None)

            @pl.when(j + 3 < NCHUNK)
            def _next():
                _issue(j + 3, 0)

            _finish(j + 2, 2)
            return carry

        lax.fori_loop(0, NT, _triple, 0)

        _drain(1, None)
        _drain(2, None)

        plsc.subcore_barrier()

        @pl.when(s < NS - 1)
        def _dump():
            pltpu.sync_copy(acc.at[pl.ds(s * RPT, RPT)],
                            s_out.at[pl.ds(c * N + s * RPT, RPT)])

        @pl.when(s == NS - 1)
        def _dump_last():
            pltpu.sync_copy(acc.at[pl.ds((NS - 1) * RPT, RPT_LAST)],
                            s_out.at[pl.ds(c * N + (NS - 1) * RPT, RPT_LAST)])

        @pl.when(c == cc)
        def _dump_cnt():
            pltpu.sync_copy(cnt.at[pl.ds(s * ZPT, ZPT)],
                            c_out.at[pl.ds(s * ZPT, ZPT)])


_sc_segsum = functools.partial(
    pl.kernel,
    out_type=[jax.ShapeDtypeStruct((2 * N, H), _f32) for _ in range(4)]
    + [jax.ShapeDtypeStruct((NACC,), _f32) for _ in range(4)],
    mesh=plsc.VectorSubcoreMesh(
        core_axis_name="c", subcore_axis_name="s", num_cores=NC, num_subcores=NS),
    scratch_types=[
        pltpu.VMEM((EPT,), _i32),
        pltpu.VMEM((CH,), _i32),
        pltpu.VMEM((CH,), _i32),
        pltpu.VMEM((CH,), _i32),
        pltpu.VMEM((CH, H), _f32),
        pltpu.VMEM((CH, H), _f32),
        pltpu.VMEM((CH, H), _f32),
        pltpu.VMEM((CH,), _f32),
        pltpu.VMEM_SHARED((NACC, H), _f32),
        pltpu.VMEM_SHARED((NACC,), _f32),
    ] + [pltpu.SemaphoreType.DMA] * 12,
)(_sc_body)


def _tc_body(xp, xg, xs,
             sgp0, sgp1, ssp0, ssp1, spg0, spg1, sps0, sps1,
             cgp, csp, cpg, cps,
             wlgp_a, wlgp_b, wlsp_a, wlsp_b, wlpg_a, wlpg_b, wlps_a, wlps_b,
             wrgp, wrsp, wrpg, wrps,
             blgp, blsp, blpg, blps,
             wgw, bgw, wsw, bsw,
             pfas_o, gw_o, sw_o):
    def dot(a, b):
        return jnp.dot(a, b, preferred_element_type=_f32)

    r_gp = 1.0 / jnp.maximum(cgp[...], 1.0)
    r_sp = 1.0 / jnp.maximum(csp[...], 1.0)
    r_pg = 1.0 / jnp.maximum(cpg[...], 1.0)
    r_ps = 1.0 / jnp.maximum(cps[...], 1.0)

    o_pf = (dot(sgp0[...] * r_gp, wlgp_a[...]) + dot(sgp1[...] * r_gp, wlgp_b[...])
            + dot(ssp0[...] * r_sp, wlsp_a[...]) + dot(ssp1[...] * r_sp, wlsp_b[...])
            + dot(xp[...], wrgp[...] + wrsp[...])
            + blgp[...] + blsp[...])
    pfas_o[...] = jnp.maximum(o_pf, 0.0)

    o_gw = (dot(spg0[...] * r_pg, wlpg_a[...]) + dot(spg1[...] * r_pg, wlpg_b[...])
            + dot(xg[...], wrpg[...]) + blpg[...])
    gw_o[...] = dot(jnp.maximum(o_gw, 0.0), wgw[...]) + bgw[...]

    o_sw = (dot(sps0[...] * r_ps, wlps_a[...]) + dot(sps1[...] * r_ps, wlps_b[...])
            + dot(xs[...], wrps[...]) + blps[...])
    sw_o[...] = dot(jnp.maximum(o_sw, 0.0), wsw[...]) + bsw[...]


_R = 1000
_G = N // _R


def _tc_dense(xp, xg, xs, s_gp, s_sp, s_pg, s_ps, c_gp, c_sp, c_pg, c_ps,
              wlgp_a, wlgp_b, wlsp_a, wlsp_b, wlpg_a, wlpg_b, wlps_a, wlps_b,
              wrgp, wrsp, wrpg, wrps, blgp, blsp, blpg, blps,
              wgw, bgw, wsw, bsw):
    row = pl.BlockSpec((_R, D), lambda i: (i, 0))
    half_lo = pl.BlockSpec((_R, H), lambda i: (i, 0))
    half_hi = pl.BlockSpec((_R, H), lambda i: (i + N // _R, 0))
    cntb = pl.BlockSpec((_R, 1), lambda i: (i, 0))
    whalf = pl.BlockSpec((H, D), lambda i: (0, 0))
    wfull = pl.BlockSpec((D, D), lambda i: (0, 0))
    brow = pl.BlockSpec((1, D), lambda i: (0, 0))
    wcol = pl.BlockSpec((D, 1), lambda i: (0, 0))
    bone = pl.BlockSpec((1, 1), lambda i: (0, 0))

    return pl.pallas_call(
        _tc_body,
        grid=(_G,),
        in_specs=[row, row, row,
                  half_lo, half_hi, half_lo, half_hi,
                  half_lo, half_hi, half_lo, half_hi,
                  cntb, cntb, cntb, cntb,
                  whalf, whalf, whalf, whalf, whalf, whalf, whalf, whalf,
                  wfull, wfull, wfull, wfull,
                  brow, brow, brow, brow,
                  wcol, bone, wcol, bone],
        out_specs=[pl.BlockSpec((_R, D), lambda i: (i, 0)),
                   pl.BlockSpec((_R, 1), lambda i: (i, 0)),
                   pl.BlockSpec((_R, 1), lambda i: (i, 0))],
        out_shape=[jax.ShapeDtypeStruct((N, D), _f32),
                   jax.ShapeDtypeStruct((N, 1), _f32),
                   jax.ShapeDtypeStruct((N, 1), _f32)],
    )(xp, xg, xs,
      s_gp, s_gp, s_sp, s_sp, s_pg, s_pg, s_ps, s_ps,
      c_gp, c_sp, c_pg, c_ps,
      wlgp_a, wlgp_b, wlsp_a, wlsp_b, wlpg_a, wlpg_b, wlps_a, wlps_b,
      wrgp, wrsp, wrpg, wrps, blgp, blsp, blpg, blps,
      wgw, bgw, wsw, bsw)


def _halves(x):
    return x.reshape(2 * N, H)


def _prep_edges(ei):
    pad = E_PAD - E
    src = jnp.concatenate([ei[0], jnp.zeros((pad,), _i32)])
    dst = jnp.concatenate([ei[1], jnp.full((pad,), N, _i32)])
    src2 = jnp.concatenate([2 * src, 2 * src + 1])
    return src2, dst


def kernel(x_pfas_sites, x_gw_wells, x_sw_stations,
           edge_index_pfas_to_gw, edge_index_gw_to_pfas,
           edge_index_pfas_to_sw, edge_index_sw_to_pfas,
           Wl_pg, bl_pg, Wr_pg,
           Wl_gp, bl_gp, Wr_gp,
           Wl_ps, bl_ps, Wr_ps,
           Wl_sp, bl_sp, Wr_sp,
           W_gw, b_gw, W_sw, b_sw):
    t_pf = _halves(x_pfas_sites)
    t_gw = _halves(x_gw_wells)
    t_sw = _halves(x_sw_stations)
    src2_pg, dst_pg = _prep_edges(edge_index_pfas_to_gw)
    src2_gp, dst_gp = _prep_edges(edge_index_gw_to_pfas)
    src2_ps, dst_ps = _prep_edges(edge_index_pfas_to_sw)
    src2_sp, dst_sp = _prep_edges(edge_index_sw_to_pfas)

    zrow = jnp.zeros((ZPT, H), _f32)
    zcnt = jnp.zeros((NACC,), _f32)
    onesh = jnp.ones((CH,), _f32)

    (s_pg, s_gp, s_ps, s_sp, c_pg, c_gp, c_ps, c_sp) = _sc_segsum(
        t_pf, t_gw, t_sw,
        src2_pg, src2_gp, src2_ps, src2_sp,
        dst_pg, dst_gp, dst_ps, dst_sp,
        zrow, zcnt, onesh)

    c_gp, c_sp, c_pg, c_ps = (c[:N].reshape(N, 1)
                              for c in (c_gp, c_sp, c_pg, c_ps))
    pfas, gw, sw = _tc_dense(
        x_pfas_sites, x_gw_wells, x_sw_stations,
        s_gp, s_sp, s_pg, s_ps, c_gp, c_sp, c_pg, c_ps,
        Wl_gp[:H], Wl_gp[H:], Wl_sp[:H], Wl_sp[H:],
        Wl_pg[:H], Wl_pg[H:], Wl_ps[:H], Wl_ps[H:],
        Wr_gp, Wr_sp, Wr_pg, Wr_ps,
        bl_gp.reshape(1, D), bl_sp.reshape(1, D),
        bl_pg.reshape(1, D), bl_ps.reshape(1, D),
        W_gw, b_gw.reshape(1, 1), W_sw, b_sw.reshape(1, 1))
    return (pfas, gw, sw)
